# baseline (speedup 1.0000x reference)
import functools

import jax
import jax.numpy as jnp
from jax import lax
from jax.experimental import pallas as pl
from jax.experimental.pallas import tpu as pltpu
from jax.experimental.pallas import tpu_sc as plsc

NUM_CORES = 2
NUM_SUBCORES = 16
NUM_WORKERS = NUM_CORES * NUM_SUBCORES
CHUNK = 128
ROW_BLK = 2048
T1_BLK = 512


def _emb_call(n_chunks, d):
    mesh = plsc.VectorSubcoreMesh(core_axis_name="c", subcore_axis_name="s")
    tok_per_worker = n_chunks * CHUNK

    @functools.partial(
        pl.kernel,
        out_type=jax.ShapeDtypeStruct((NUM_WORKERS * tok_per_worker, d),
                                      jnp.float32),
        mesh=mesh,
        scratch_types=[
            pltpu.VMEM((n_chunks, CHUNK), jnp.int32),
            pltpu.VMEM((2, CHUNK, d), jnp.float32),
            pltpu.SemaphoreType.DMA,
            pltpu.SemaphoreType.DMA,
            pltpu.SemaphoreType.DMA,
            pltpu.SemaphoreType.DMA,
        ],
        compiler_params=pltpu.CompilerParams(use_tc_tiling_on_sc=False),
    )
    def emb(idx_hbm, w_hbm, out_hbm, idx_v, rows_v, g0, g1, w0, w1):
        wid = lax.axis_index("s") * NUM_CORES + lax.axis_index("c")
        base = wid * tok_per_worker
        pltpu.sync_copy(idx_hbm.at[wid], idx_v)
        gsems = (g0, g1)
        wsems = (w0, w1)

        def fire(g, buf):
            pltpu.async_copy(
                w_hbm.at[idx_v.at[g]], rows_v.at[buf], gsems[buf])

        def drain_gather(buf):
            pltpu.make_async_copy(
                out_hbm.at[pl.ds(0, CHUNK)], rows_v.at[buf],
                gsems[buf]).wait()

        def wait_writeback(g, buf):
            pltpu.make_async_copy(
                rows_v.at[buf],
                out_hbm.at[pl.ds(base + g * CHUNK, CHUNK)],
                wsems[buf]).wait()

        def do_step(g, buf):
            drain_gather(buf)
            pltpu.async_copy(
                rows_v.at[buf],
                out_hbm.at[pl.ds(base + g * CHUNK, CHUNK)],
                wsems[buf])
            nxt = buf ^ 1

            @pl.when(g > 0)
            def _():
                wait_writeback(g - 1, nxt)

            @pl.when(g + 1 < n_chunks)
            def _():
                fire(g + 1, nxt)

        fire(0, 0)

        def body(i, carry):
            do_step(2 * i, 0)
            do_step(2 * i + 1, 1)
            return carry

        lax.fori_loop(0, n_chunks // 2, body, 0)
        wait_writeback(n_chunks - 1, (n_chunks - 1) % 2)

    return emb


def _pack_rows(wt, n_rows, d):
    n_blk = (n_rows + ROW_BLK - 1) // ROW_BLK
    half = ROW_BLK // 2

    def body(in_ref, out_ref):
        pairs = in_ref[...].reshape(d, half, 2)
        out_ref[...] = jnp.transpose(pairs, (1, 2, 0)).reshape(half, 2 * d)

    return pl.pallas_call(
        body,
        grid=(n_blk,),
        in_specs=[pl.BlockSpec((d, ROW_BLK), lambda i: (0, i))],
        out_specs=pl.BlockSpec((half, 2 * d), lambda i: (i, 0)),
        out_shape=jax.ShapeDtypeStruct((n_blk * half, 2 * d), jnp.float32),
    )(wt)


def _finalize(out2, n_seq, seq_len, d):
    n_blk = n_seq // T1_BLK
    lines = T1_BLK * seq_len // 2

    def body(in_ref, out_ref):
        xr = in_ref[...].reshape(T1_BLK, seq_len // 2, 2 * d)
        for t2 in range(seq_len):
            c0 = (t2 % 2) * d
            out_ref[t2] = xr[:, t2 // 2, c0:c0 + d].T

    return pl.pallas_call(
        body,
        grid=(n_blk,),
        in_specs=[pl.BlockSpec((lines, 2 * d), lambda i: (i, 0))],
        out_specs=pl.BlockSpec((seq_len, d, T1_BLK), lambda i: (0, 0, i)),
        out_shape=jax.ShapeDtypeStruct((seq_len, d, n_seq), jnp.float32),
    )(out2)


def kernel(x, weight):
    n_seq, seq_len = x.shape
    n_rows, d = weight.shape
    n_tok = n_seq * seq_len
    assert n_tok % (NUM_WORKERS * CHUNK) == 0 and seq_len % 2 == 0
    n_chunks = n_tok // (NUM_WORKERS * CHUNK)
    idx = x.reshape(NUM_WORKERS, n_chunks, CHUNK).astype(jnp.int32)
    wpacked = _pack_rows(weight.T, n_rows, d)
    w2d = wpacked.reshape(wpacked.shape[0] * 2, d)
    out2d = _emb_call(n_chunks, d)(idx, w2d)
    out2 = out2d.reshape(n_tok // 2, 2 * d)
    o3 = _finalize(out2, n_seq, seq_len, d)
    return jnp.transpose(o3, (2, 0, 1))


# speedup vs baseline: 13.1571x; 13.1571x over previous
import functools

import jax
import jax.numpy as jnp
from jax import lax
from jax.experimental import pallas as pl
from jax.experimental.pallas import tpu as pltpu
from jax.experimental.pallas import tpu_sc as plsc

NUM_CORES = 2
NUM_SUBCORES = 16
NUM_WORKERS = NUM_CORES * NUM_SUBCORES
CHUNK = 128
ROW_BLK = 2048
T1_BLK = 512


def _emb_call(n_chunks, d):
    mesh = plsc.VectorSubcoreMesh(core_axis_name="c", subcore_axis_name="s")
    tok_per_worker = n_chunks * CHUNK

    @functools.partial(
        pl.kernel,
        out_type=jax.ShapeDtypeStruct((NUM_WORKERS * tok_per_worker, d),
                                      jnp.float32),
        mesh=mesh,
        scratch_types=[
            pltpu.VMEM((n_chunks, CHUNK), jnp.int32),
            pltpu.VMEM((2, CHUNK, d), jnp.float32),
            pltpu.SemaphoreType.DMA,
            pltpu.SemaphoreType.DMA,
            pltpu.SemaphoreType.DMA,
            pltpu.SemaphoreType.DMA,
        ],
        compiler_params=pltpu.CompilerParams(use_tc_tiling_on_sc=False),
    )
    def emb(idx_hbm, w_hbm, out_hbm, idx_v, rows_v, g0, g1, w0, w1):
        wid = lax.axis_index("s") * NUM_CORES + lax.axis_index("c")
        base = wid * tok_per_worker
        pltpu.sync_copy(idx_hbm.at[wid], idx_v)
        gsems = (g0, g1)
        wsems = (w0, w1)

        def fire(g, buf):
            pltpu.async_copy(
                w_hbm.at[idx_v.at[g]], rows_v.at[buf], gsems[buf])

        def drain_gather(buf):
            pltpu.make_async_copy(
                out_hbm.at[pl.ds(0, CHUNK)], rows_v.at[buf],
                gsems[buf]).wait()

        def wait_writeback(g, buf):
            pltpu.make_async_copy(
                rows_v.at[buf],
                out_hbm.at[pl.ds(base + g * CHUNK, CHUNK)],
                wsems[buf]).wait()

        def do_step(g, buf):
            drain_gather(buf)
            pltpu.async_copy(
                rows_v.at[buf],
                out_hbm.at[pl.ds(base + g * CHUNK, CHUNK)],
                wsems[buf])
            nxt = buf ^ 1

            @pl.when(g > 0)
            def _():
                wait_writeback(g - 1, nxt)

            @pl.when(g + 1 < n_chunks)
            def _():
                fire(g + 1, nxt)

        fire(0, 0)

        def body(i, carry):
            do_step(2 * i, 0)
            do_step(2 * i + 1, 1)
            return carry

        lax.fori_loop(0, n_chunks // 2, body, 0)
        wait_writeback(n_chunks - 1, (n_chunks - 1) % 2)

    return emb


PACK_BLK = 1024


def _pack_rows(wt, n_rows, d):
    n_blk = (n_rows + 2 * PACK_BLK - 1) // (2 * PACK_BLK)
    half = n_blk * PACK_BLK
    hi_max = (n_rows + PACK_BLK - 1) // PACK_BLK - 1

    def body(lo_ref, hi_ref, out_ref):
        out_ref[:, 0:d] = lo_ref[...].T
        out_ref[:, d:2 * d] = hi_ref[...].T

    return half, pl.pallas_call(
        body,
        grid=(n_blk,),
        in_specs=[pl.BlockSpec((d, PACK_BLK), lambda i: (0, i)),
                  pl.BlockSpec((d, PACK_BLK),
                               lambda i: (0, jnp.minimum(i + n_blk, hi_max)))],
        out_specs=pl.BlockSpec((PACK_BLK, 2 * d), lambda i: (i, 0)),
        out_shape=jax.ShapeDtypeStruct((half, 2 * d), jnp.float32),
    )(wt, wt)


def _finalize(out2, n_seq, seq_len, d):
    n_blk = n_seq // T1_BLK
    lines = T1_BLK * seq_len // 2

    def body(in_ref, out_ref):
        xr = in_ref[...].reshape(T1_BLK, seq_len // 2, 2 * d)
        for t2 in range(seq_len):
            c0 = (t2 % 2) * d
            out_ref[t2] = xr[:, t2 // 2, c0:c0 + d].T

    return pl.pallas_call(
        body,
        grid=(n_blk,),
        in_specs=[pl.BlockSpec((lines, 2 * d), lambda i: (i, 0))],
        out_specs=pl.BlockSpec((seq_len, d, T1_BLK), lambda i: (0, 0, i)),
        out_shape=jax.ShapeDtypeStruct((seq_len, d, n_seq), jnp.float32),
    )(out2)


def kernel(x, weight):
    n_seq, seq_len = x.shape
    n_rows, d = weight.shape
    n_tok = n_seq * seq_len
    assert n_tok % (NUM_WORKERS * CHUNK) == 0 and seq_len % 2 == 0
    n_chunks = n_tok // (NUM_WORKERS * CHUNK)
    half, wpacked = _pack_rows(weight.T, n_rows, d)
    x32 = x.astype(jnp.int32)
    remapped = jnp.where(x32 < half, 2 * x32, 2 * (x32 - half) + 1)
    idx = remapped.reshape(NUM_WORKERS, n_chunks, CHUNK)
    w2d = wpacked.reshape(half * 2, d)
    out2d = _emb_call(n_chunks, d)(idx, w2d)
    out2 = out2d.reshape(n_tok // 2, 2 * d)
    o3 = _finalize(out2, n_seq, seq_len, d)
    return jnp.transpose(o3, (2, 0, 1))


# speedup vs baseline: 15.9857x; 1.2150x over previous
import functools

import jax
import jax.numpy as jnp
from jax import lax
from jax.experimental import pallas as pl
from jax.experimental.pallas import tpu as pltpu
from jax.experimental.pallas import tpu_sc as plsc

NUM_CORES = 2
NUM_SUBCORES = 16
NUM_WORKERS = NUM_CORES * NUM_SUBCORES
CHUNK = 128
ROW_BLK = 2048
T1_BLK = 1024


def _emb_call(n_chunks, d):
    mesh = plsc.VectorSubcoreMesh(core_axis_name="c", subcore_axis_name="s")
    tok_per_worker = n_chunks * CHUNK

    @functools.partial(
        pl.kernel,
        out_type=jax.ShapeDtypeStruct((NUM_WORKERS * tok_per_worker, d),
                                      jnp.float32),
        mesh=mesh,
        scratch_types=[
            pltpu.VMEM((n_chunks, CHUNK), jnp.int32),
            pltpu.VMEM((2, CHUNK, d), jnp.float32),
            pltpu.SemaphoreType.DMA,
            pltpu.SemaphoreType.DMA,
            pltpu.SemaphoreType.DMA,
            pltpu.SemaphoreType.DMA,
        ],
        compiler_params=pltpu.CompilerParams(use_tc_tiling_on_sc=False),
    )
    def emb(idx_hbm, w_hbm, out_hbm, idx_v, rows_v, g0, g1, w0, w1):
        wid = lax.axis_index("s") * NUM_CORES + lax.axis_index("c")
        base = wid * tok_per_worker
        pltpu.sync_copy(idx_hbm.at[wid], idx_v)
        gsems = (g0, g1)
        wsems = (w0, w1)

        def fire(g, buf):
            pltpu.async_copy(
                w_hbm.at[idx_v.at[g]], rows_v.at[buf], gsems[buf])

        def drain_gather(buf):
            pltpu.make_async_copy(
                out_hbm.at[pl.ds(0, CHUNK)], rows_v.at[buf],
                gsems[buf]).wait()

        def wait_writeback(g, buf):
            pltpu.make_async_copy(
                rows_v.at[buf],
                out_hbm.at[pl.ds(base + g * CHUNK, CHUNK)],
                wsems[buf]).wait()

        def do_step(g, buf):
            drain_gather(buf)
            pltpu.async_copy(
                rows_v.at[buf],
                out_hbm.at[pl.ds(base + g * CHUNK, CHUNK)],
                wsems[buf])
            nxt = buf ^ 1

            @pl.when(g > 0)
            def _():
                wait_writeback(g - 1, nxt)

            @pl.when(g + 1 < n_chunks)
            def _():
                fire(g + 1, nxt)

        fire(0, 0)

        def body(i, carry):
            do_step(2 * i, 0)
            do_step(2 * i + 1, 1)
            return carry

        lax.fori_loop(0, n_chunks // 2, body, 0)
        wait_writeback(n_chunks - 1, (n_chunks - 1) % 2)

    return emb


PACK_BLK = 2048


def _pack_rows(wt, n_rows, d):
    n_blk = (n_rows + 2 * PACK_BLK - 1) // (2 * PACK_BLK)
    half = n_blk * PACK_BLK
    hi_max = (n_rows + PACK_BLK - 1) // PACK_BLK - 1

    def body(lo_ref, hi_ref, out_ref):
        out_ref[:, 0:d] = lo_ref[...].T
        out_ref[:, d:2 * d] = hi_ref[...].T

    return half, pl.pallas_call(
        body,
        grid=(n_blk,),
        in_specs=[pl.BlockSpec((d, PACK_BLK), lambda i: (0, i)),
                  pl.BlockSpec((d, PACK_BLK),
                               lambda i: (0, jnp.minimum(i + n_blk, hi_max)))],
        out_specs=pl.BlockSpec((PACK_BLK, 2 * d), lambda i: (i, 0)),
        out_shape=jax.ShapeDtypeStruct((half, 2 * d), jnp.float32),
    )(wt, wt)


def _finalize(out2, n_seq, seq_len, d):
    n_blk = n_seq // T1_BLK
    lines = T1_BLK * seq_len // 2

    def body(in_ref, out_ref):
        xr = in_ref[...].reshape(T1_BLK, seq_len // 2, 2 * d)
        for t2 in range(seq_len):
            c0 = (t2 % 2) * d
            out_ref[t2] = xr[:, t2 // 2, c0:c0 + d].T

    return pl.pallas_call(
        body,
        grid=(n_blk,),
        in_specs=[pl.BlockSpec((lines, 2 * d), lambda i: (i, 0))],
        out_specs=pl.BlockSpec((seq_len, d, T1_BLK), lambda i: (0, 0, i)),
        out_shape=jax.ShapeDtypeStruct((seq_len, d, n_seq), jnp.float32),
    )(out2)


def kernel(x, weight):
    n_seq, seq_len = x.shape
    n_rows, d = weight.shape
    n_tok = n_seq * seq_len
    assert n_tok % (NUM_WORKERS * CHUNK) == 0 and seq_len % 2 == 0
    n_chunks = n_tok // (NUM_WORKERS * CHUNK)
    half, wpacked = _pack_rows(weight.T, n_rows, d)
    x32 = x.astype(jnp.int32)
    remapped = jnp.where(x32 < half, 2 * x32, 2 * (x32 - half) + 1)
    idx = remapped.reshape(NUM_WORKERS, n_chunks, CHUNK)
    w2d = wpacked.reshape(half * 2, d)
    out2d = _emb_call(n_chunks, d)(idx, w2d)
    out2 = out2d.reshape(n_tok // 2, 2 * d)
    o3 = _finalize(out2, n_seq, seq_len, d)
    return jnp.transpose(o3, (2, 0, 1))


# speedup vs baseline: 18.1582x; 1.1359x over previous
import functools

import jax
import jax.numpy as jnp
from jax import lax
from jax.experimental import pallas as pl
from jax.experimental.pallas import tpu as pltpu
from jax.experimental.pallas import tpu_sc as plsc

NUM_CORES = 2
NUM_SUBCORES = 16
NUM_WORKERS = NUM_CORES * NUM_SUBCORES
CHUNK = 128
ROW_BLK = 2048
T1_BLK = 1024


def _emb_call(n_chunks, d):
    mesh = plsc.VectorSubcoreMesh(core_axis_name="c", subcore_axis_name="s")
    tok_per_worker = n_chunks * CHUNK

    @functools.partial(
        pl.kernel,
        out_type=jax.ShapeDtypeStruct((NUM_WORKERS * tok_per_worker, d),
                                      jnp.float32),
        mesh=mesh,
        scratch_types=[
            pltpu.VMEM((n_chunks, CHUNK), jnp.int32),
            pltpu.VMEM((2, CHUNK, d), jnp.float32),
            pltpu.SemaphoreType.DMA,
            pltpu.SemaphoreType.DMA,
            pltpu.SemaphoreType.DMA,
            pltpu.SemaphoreType.DMA,
        ],
        compiler_params=pltpu.CompilerParams(use_tc_tiling_on_sc=False),
    )
    def emb(idx_hbm, w_hbm, out_hbm, idx_v, rows_v, g0, g1, w0, w1):
        wid = lax.axis_index("s") * NUM_CORES + lax.axis_index("c")
        base = wid * tok_per_worker
        pltpu.sync_copy(idx_hbm.at[wid], idx_v)
        gsems = (g0, g1)
        wsems = (w0, w1)

        def fire(g, buf):
            pltpu.async_copy(
                w_hbm.at[idx_v.at[g]], rows_v.at[buf], gsems[buf])

        def drain_gather(buf):
            pltpu.make_async_copy(
                out_hbm.at[pl.ds(0, CHUNK)], rows_v.at[buf],
                gsems[buf]).wait()

        def wait_writeback(g, buf):
            pltpu.make_async_copy(
                rows_v.at[buf],
                out_hbm.at[pl.ds(base + g * CHUNK, CHUNK)],
                wsems[buf]).wait()

        def do_step(g, buf):
            drain_gather(buf)
            pltpu.async_copy(
                rows_v.at[buf],
                out_hbm.at[pl.ds(base + g * CHUNK, CHUNK)],
                wsems[buf])
            nxt = buf ^ 1

            @pl.when(g > 0)
            def _():
                wait_writeback(g - 1, nxt)

            @pl.when(g + 1 < n_chunks)
            def _():
                fire(g + 1, nxt)

        fire(0, 0)

        def body(i, carry):
            do_step(2 * i, 0)
            do_step(2 * i + 1, 1)
            return carry

        lax.fori_loop(0, n_chunks // 2, body, 0)
        wait_writeback(n_chunks - 1, (n_chunks - 1) % 2)

    return emb


PACK_BLK = 4096


def _pack_rows(wt, n_rows, d):
    n_blk = (n_rows + 2 * PACK_BLK - 1) // (2 * PACK_BLK)
    half = n_blk * PACK_BLK
    hi_max = (n_rows + PACK_BLK - 1) // PACK_BLK - 1

    def body(lo_ref, hi_ref, out_ref):
        out_ref[:, 0:d] = lo_ref[...].T
        out_ref[:, d:2 * d] = hi_ref[...].T

    return half, pl.pallas_call(
        body,
        grid=(n_blk,),
        in_specs=[pl.BlockSpec((d, PACK_BLK), lambda i: (0, i)),
                  pl.BlockSpec((d, PACK_BLK),
                               lambda i: (0, jnp.minimum(i + n_blk, hi_max)))],
        out_specs=pl.BlockSpec((PACK_BLK, 2 * d), lambda i: (i, 0)),
        out_shape=jax.ShapeDtypeStruct((half, 2 * d), jnp.float32),
    )(wt, wt)


def _finalize(out2, n_seq, seq_len, d):
    n_blk = n_seq // T1_BLK
    lines = T1_BLK * seq_len // 2

    def body(in_ref, out_ref):
        xr = in_ref[...].reshape(T1_BLK, seq_len // 2, 2 * d)
        for t2 in range(seq_len):
            c0 = (t2 % 2) * d
            out_ref[t2] = xr[:, t2 // 2, c0:c0 + d].T

    return pl.pallas_call(
        body,
        grid=(n_blk,),
        in_specs=[pl.BlockSpec((lines, 2 * d), lambda i: (i, 0))],
        out_specs=pl.BlockSpec((seq_len, d, T1_BLK), lambda i: (0, 0, i)),
        out_shape=jax.ShapeDtypeStruct((seq_len, d, n_seq), jnp.float32),
    )(out2)


def kernel(x, weight):
    n_seq, seq_len = x.shape
    n_rows, d = weight.shape
    n_tok = n_seq * seq_len
    assert n_tok % (NUM_WORKERS * CHUNK) == 0 and seq_len % 2 == 0
    n_chunks = n_tok // (NUM_WORKERS * CHUNK)
    half, wpacked = _pack_rows(weight.T, n_rows, d)
    x32 = x.astype(jnp.int32)
    remapped = jnp.where(x32 < half, 2 * x32, 2 * (x32 - half) + 1)
    idx = remapped.reshape(NUM_WORKERS, n_chunks, CHUNK)
    w2d = wpacked.reshape(half * 2, d)
    out2d = _emb_call(n_chunks, d)(idx, w2d)
    out2 = out2d.reshape(n_tok // 2, 2 * d)
    o3 = _finalize(out2, n_seq, seq_len, d)
    return jnp.transpose(o3, (2, 0, 1))


# speedup vs baseline: 19.4126x; 1.0691x over previous
import functools

import jax
import jax.numpy as jnp
from jax import lax
from jax.experimental import pallas as pl
from jax.experimental.pallas import tpu as pltpu
from jax.experimental.pallas import tpu_sc as plsc

NUM_CORES = 2
NUM_SUBCORES = 16
NUM_WORKERS = NUM_CORES * NUM_SUBCORES
CHUNK = 128
ROW_BLK = 2048
T1_BLK = 1024


def _emb_call(n_chunks, d):
    mesh = plsc.VectorSubcoreMesh(core_axis_name="c", subcore_axis_name="s")
    tok_per_worker = n_chunks * CHUNK

    @functools.partial(
        pl.kernel,
        out_type=jax.ShapeDtypeStruct((NUM_WORKERS * tok_per_worker, d),
                                      jnp.float32),
        mesh=mesh,
        scratch_types=[
            pltpu.VMEM((n_chunks, CHUNK), jnp.int32),
            pltpu.VMEM((2, CHUNK, d), jnp.float32),
            pltpu.SemaphoreType.DMA,
            pltpu.SemaphoreType.DMA,
            pltpu.SemaphoreType.DMA,
            pltpu.SemaphoreType.DMA,
        ],
        compiler_params=pltpu.CompilerParams(use_tc_tiling_on_sc=False),
    )
    def emb(idx_hbm, w_hbm, out_hbm, idx_v, rows_v, g0, g1, w0, w1):
        wid = lax.axis_index("s") * NUM_CORES + lax.axis_index("c")
        base = wid * tok_per_worker
        pltpu.sync_copy(idx_hbm.at[wid], idx_v)
        gsems = (g0, g1)
        wsems = (w0, w1)

        def fire(g, buf):
            pltpu.async_copy(
                w_hbm.at[idx_v.at[g]], rows_v.at[buf], gsems[buf])

        def drain_gather(buf):
            pltpu.make_async_copy(
                out_hbm.at[pl.ds(0, CHUNK)], rows_v.at[buf],
                gsems[buf]).wait()

        def wait_writeback(g, buf):
            pltpu.make_async_copy(
                rows_v.at[buf],
                out_hbm.at[pl.ds(base + g * CHUNK, CHUNK)],
                wsems[buf]).wait()

        def do_step(g, buf):
            drain_gather(buf)
            pltpu.async_copy(
                rows_v.at[buf],
                out_hbm.at[pl.ds(base + g * CHUNK, CHUNK)],
                wsems[buf])
            nxt = buf ^ 1

            @pl.when(g > 0)
            def _():
                wait_writeback(g - 1, nxt)

            @pl.when(g + 1 < n_chunks)
            def _():
                fire(g + 1, nxt)

        fire(0, 0)

        def body(i, carry):
            do_step(2 * i, 0)
            do_step(2 * i + 1, 1)
            return carry

        lax.fori_loop(0, n_chunks // 2, body, 0)
        wait_writeback(n_chunks - 1, (n_chunks - 1) % 2)

    return emb


PACK_BLK = 8192


def _pack_rows(wt, n_rows, d):
    n_blk = (n_rows + 2 * PACK_BLK - 1) // (2 * PACK_BLK)
    half = n_blk * PACK_BLK
    hi_max = (n_rows + PACK_BLK - 1) // PACK_BLK - 1

    def body(lo_ref, hi_ref, out_ref):
        out_ref[:, 0:d] = lo_ref[...].T
        out_ref[:, d:2 * d] = hi_ref[...].T

    return half, pl.pallas_call(
        body,
        grid=(n_blk,),
        in_specs=[pl.BlockSpec((d, PACK_BLK), lambda i: (0, i)),
                  pl.BlockSpec((d, PACK_BLK),
                               lambda i: (0, jnp.minimum(i + n_blk, hi_max)))],
        out_specs=pl.BlockSpec((PACK_BLK, 2 * d), lambda i: (i, 0)),
        out_shape=jax.ShapeDtypeStruct((half, 2 * d), jnp.float32),
    )(wt, wt)


def _finalize(out2, n_seq, seq_len, d):
    n_blk = n_seq // T1_BLK
    lines = T1_BLK * seq_len // 2

    def body(in_ref, out_ref):
        xr = in_ref[...].reshape(T1_BLK, seq_len // 2, 2 * d)
        for t2 in range(seq_len):
            c0 = (t2 % 2) * d
            out_ref[t2] = xr[:, t2 // 2, c0:c0 + d].T

    return pl.pallas_call(
        body,
        grid=(n_blk,),
        in_specs=[pl.BlockSpec((lines, 2 * d), lambda i: (i, 0))],
        out_specs=pl.BlockSpec((seq_len, d, T1_BLK), lambda i: (0, 0, i)),
        out_shape=jax.ShapeDtypeStruct((seq_len, d, n_seq), jnp.float32),
    )(out2)


def kernel(x, weight):
    n_seq, seq_len = x.shape
    n_rows, d = weight.shape
    n_tok = n_seq * seq_len
    assert n_tok % (NUM_WORKERS * CHUNK) == 0 and seq_len % 2 == 0
    n_chunks = n_tok // (NUM_WORKERS * CHUNK)
    half, wpacked = _pack_rows(weight.T, n_rows, d)
    x32 = x.astype(jnp.int32)
    remapped = jnp.where(x32 < half, 2 * x32, 2 * (x32 - half) + 1)
    idx = remapped.reshape(NUM_WORKERS, n_chunks, CHUNK)
    w2d = wpacked.reshape(half * 2, d)
    out2d = _emb_call(n_chunks, d)(idx, w2d)
    out2 = out2d.reshape(n_tok // 2, 2 * d)
    o3 = _finalize(out2, n_seq, seq_len, d)
    return jnp.transpose(o3, (2, 0, 1))


# speedup vs baseline: 20.0662x; 1.0337x over previous
import functools

import jax
import jax.numpy as jnp
from jax import lax
from jax.experimental import pallas as pl
from jax.experimental.pallas import tpu as pltpu
from jax.experimental.pallas import tpu_sc as plsc

NUM_CORES = 2
NUM_SUBCORES = 16
NUM_WORKERS = NUM_CORES * NUM_SUBCORES
CHUNK = 128
ROW_BLK = 2048
T1_BLK = 1024


def _emb_call(n_chunks, d):
    mesh = plsc.VectorSubcoreMesh(core_axis_name="c", subcore_axis_name="s")
    tok_per_worker = n_chunks * CHUNK

    @functools.partial(
        pl.kernel,
        out_type=jax.ShapeDtypeStruct((NUM_WORKERS * tok_per_worker, d),
                                      jnp.float32),
        mesh=mesh,
        scratch_types=[
            pltpu.VMEM((n_chunks, CHUNK), jnp.int32),
            pltpu.VMEM((2, CHUNK, d), jnp.float32),
            pltpu.SemaphoreType.DMA,
            pltpu.SemaphoreType.DMA,
            pltpu.SemaphoreType.DMA,
            pltpu.SemaphoreType.DMA,
        ],
        compiler_params=pltpu.CompilerParams(use_tc_tiling_on_sc=False),
    )
    def emb(idx_hbm, w_hbm, out_hbm, idx_v, rows_v, g0, g1, w0, w1):
        wid = lax.axis_index("s") * NUM_CORES + lax.axis_index("c")
        base = wid * tok_per_worker
        pltpu.sync_copy(idx_hbm.at[wid], idx_v)
        gsems = (g0, g1)
        wsems = (w0, w1)

        def fire(g, buf):
            pltpu.async_copy(
                w_hbm.at[idx_v.at[g]], rows_v.at[buf], gsems[buf])

        def drain_gather(buf):
            pltpu.make_async_copy(
                out_hbm.at[pl.ds(0, CHUNK)], rows_v.at[buf],
                gsems[buf]).wait()

        def wait_writeback(g, buf):
            pltpu.make_async_copy(
                rows_v.at[buf],
                out_hbm.at[pl.ds(base + g * CHUNK, CHUNK)],
                wsems[buf]).wait()

        def do_step(g, buf):
            drain_gather(buf)
            pltpu.async_copy(
                rows_v.at[buf],
                out_hbm.at[pl.ds(base + g * CHUNK, CHUNK)],
                wsems[buf])
            nxt = buf ^ 1

            @pl.when(g > 0)
            def _():
                wait_writeback(g - 1, nxt)

            @pl.when(g + 1 < n_chunks)
            def _():
                fire(g + 1, nxt)

        fire(0, 0)

        def body(i, carry):
            do_step(2 * i, 0)
            do_step(2 * i + 1, 1)
            return carry

        lax.fori_loop(0, n_chunks // 2, body, 0)
        wait_writeback(n_chunks - 1, (n_chunks - 1) % 2)

    return emb


PACK_BLK = 16384


def _pack_rows(wt, n_rows, d):
    n_blk = (n_rows + 2 * PACK_BLK - 1) // (2 * PACK_BLK)
    half = n_blk * PACK_BLK
    hi_max = (n_rows + PACK_BLK - 1) // PACK_BLK - 1

    def body(lo_ref, hi_ref, out_ref):
        out_ref[:, 0:d] = lo_ref[...].T
        out_ref[:, d:2 * d] = hi_ref[...].T

    return half, pl.pallas_call(
        body,
        grid=(n_blk,),
        in_specs=[pl.BlockSpec((d, PACK_BLK), lambda i: (0, i)),
                  pl.BlockSpec((d, PACK_BLK),
                               lambda i: (0, jnp.minimum(i + n_blk, hi_max)))],
        out_specs=pl.BlockSpec((PACK_BLK, 2 * d), lambda i: (i, 0)),
        out_shape=jax.ShapeDtypeStruct((half, 2 * d), jnp.float32),
    )(wt, wt)


def _finalize(out2, n_seq, seq_len, d):
    n_blk = n_seq // T1_BLK
    lines = T1_BLK * seq_len // 2

    def body(in_ref, out_ref):
        xr = in_ref[...].reshape(T1_BLK, seq_len // 2, 2 * d)
        for t2 in range(seq_len):
            c0 = (t2 % 2) * d
            out_ref[t2] = xr[:, t2 // 2, c0:c0 + d].T

    return pl.pallas_call(
        body,
        grid=(n_blk,),
        in_specs=[pl.BlockSpec((lines, 2 * d), lambda i: (i, 0))],
        out_specs=pl.BlockSpec((seq_len, d, T1_BLK), lambda i: (0, 0, i)),
        out_shape=jax.ShapeDtypeStruct((seq_len, d, n_seq), jnp.float32),
    )(out2)


def kernel(x, weight):
    n_seq, seq_len = x.shape
    n_rows, d = weight.shape
    n_tok = n_seq * seq_len
    assert n_tok % (NUM_WORKERS * CHUNK) == 0 and seq_len % 2 == 0
    n_chunks = n_tok // (NUM_WORKERS * CHUNK)
    half, wpacked = _pack_rows(weight.T, n_rows, d)
    x32 = x.astype(jnp.int32)
    remapped = jnp.where(x32 < half, 2 * x32, 2 * (x32 - half) + 1)
    idx = remapped.reshape(NUM_WORKERS, n_chunks, CHUNK)
    w2d = wpacked.reshape(half * 2, d)
    out2d = _emb_call(n_chunks, d)(idx, w2d)
    out2 = out2d.reshape(n_tok // 2, 2 * d)
    o3 = _finalize(out2, n_seq, seq_len, d)
    return jnp.transpose(o3, (2, 0, 1))
